# baseline (speedup 1.0000x reference)
import functools

import jax
import jax.numpy as jnp
from jax import lax
from jax.experimental import pallas as pl
from jax.experimental.pallas import tpu as pltpu
from jax.experimental.pallas import tpu_sc as plsc

B = 16384
DIM = 64
RATE_REG = 0.0001
LANES = 16
NC, NS = 2, 16
NW = NC * NS
BPW = B // NW
GCHUNK = 128
NCH = BPW // GCHUNK


def _sc_kernel(users_hbm, pos_hbm, neg_hbm, eu_hbm, ei_hbm,
               d_out, sq_out,
               idx_u, idx_p, idx_n, rows_u, rows_p, rows_n,
               d_part, sq_acc, sems):
    wid = lax.axis_index("s") * NC + lax.axis_index("c")
    base = wid * BPW

    pltpu.sync_copy(users_hbm.at[pl.ds(base, BPW)], idx_u)
    pltpu.sync_copy(pos_hbm.at[pl.ds(base, BPW)], idx_p)
    pltpu.sync_copy(neg_hbm.at[pl.ds(base, BPW)], idx_n)

    copies = []
    for j in range(NCH):
        sl = pl.ds(j * GCHUNK, GCHUNK)
        copies.append(pltpu.async_copy(
            eu_hbm.at[idx_u.at[sl]], rows_u.at[sl], sems.at[0]))
        copies.append(pltpu.async_copy(
            ei_hbm.at[idx_p.at[sl]], rows_p.at[sl], sems.at[1]))
        copies.append(pltpu.async_copy(
            ei_hbm.at[idx_n.at[sl]], rows_n.at[sl], sems.at[2]))
    for c in copies:
        c.wait()

    sq_acc[...] = jnp.zeros((LANES,), jnp.float32)

    @pl.loop(0, BPW)
    def _(i):
        d_vec = None
        s_vec = None
        for c in range(DIM // LANES):
            sl = pl.ds(c * LANES, LANES)
            u = rows_u[i, sl]
            p = rows_p[i, sl]
            n = rows_n[i, sl]
            d_c = u * (p - n)
            s_c = u * u + (p * p + n * n)
            d_vec = d_c if d_vec is None else d_vec + d_c
            s_vec = s_c if s_vec is None else s_vec + s_c
        d_part[i, :] = d_vec
        sq_acc[...] = sq_acc[...] + s_vec

    pltpu.sync_copy(d_part, d_out.at[pl.ds(base, BPW)])
    pltpu.sync_copy(sq_acc, sq_out.at[wid])


def _sc_gather_partials(users, items_pos, items_neg, embed_user, embed_item):
    mesh = plsc.VectorSubcoreMesh(core_axis_name="c", subcore_axis_name="s")
    kern = functools.partial(
        pl.kernel,
        mesh=mesh,
        compiler_params=pltpu.CompilerParams(use_tc_tiling_on_sc=False),
        out_type=(
            jax.ShapeDtypeStruct((B, LANES), jnp.float32),
            jax.ShapeDtypeStruct((NW, LANES), jnp.float32),
        ),
        scratch_types=[
            pltpu.VMEM((BPW,), jnp.int32),
            pltpu.VMEM((BPW,), jnp.int32),
            pltpu.VMEM((BPW,), jnp.int32),
            pltpu.VMEM((BPW, DIM), jnp.float32),
            pltpu.VMEM((BPW, DIM), jnp.float32),
            pltpu.VMEM((BPW, DIM), jnp.float32),
            pltpu.VMEM((BPW, LANES), jnp.float32),
            pltpu.VMEM((LANES,), jnp.float32),
            pltpu.SemaphoreType.DMA((3,)),
        ],
    )(_sc_kernel)
    return kern(users, items_pos, items_neg, embed_user, embed_item)


def _finish_body(d_ref, sq_ref, base_ref, reg_ref):
    d = jnp.sum(d_ref[...], axis=1)
    sp = jnp.maximum(-d, 0.0) + jnp.log1p(jnp.exp(-jnp.abs(d)))
    base_ref[0, 0] = jnp.sum(sp) * (1.0 / B)
    reg_ref[0, 0] = (0.5 * RATE_REG) * jnp.sum(sq_ref[...])


def _tc_finish(d_part, sq_part):
    return pl.pallas_call(
        _finish_body,
        out_shape=(
            jax.ShapeDtypeStruct((1, 1), jnp.float32),
            jax.ShapeDtypeStruct((1, 1), jnp.float32),
        ),
        out_specs=(
            pl.BlockSpec(memory_space=pltpu.SMEM),
            pl.BlockSpec(memory_space=pltpu.SMEM),
        ),
    )(d_part, sq_part)


def kernel(users, items_pos, items_neg, embed_user, embed_item):
    d_part, sq_part = _sc_gather_partials(
        users, items_pos, items_neg, embed_user, embed_item)
    base2d, reg2d = _tc_finish(d_part, sq_part)
    return base2d[0, 0], reg2d[0, 0]
